# baseline (speedup 1.0000x reference)
import functools

import jax
import jax.numpy as jnp
from jax import lax
from jax.experimental import pallas as pl
from jax.experimental.pallas import tpu as pltpu
from jax.experimental.pallas import tpu_sc as plsc

_NUM_TAGS = 3
_NUM_CLASSES = 8
_TAG_START = 46
_TASK_START = 49
_VALUE_BASE = 22
_LOGIT_SCALE = 12.0


def _body(tok_ref, tbl_ref, out_ref):
    t = tok_ref[...]
    Bb, S = t.shape
    pos = jax.lax.broadcasted_iota(jnp.int32, (Bb, S), 1)
    is_tag = (t >= _TAG_START) & (t < _TAG_START + _NUM_TAGS)
    key = jnp.where(is_tag, pos * 4 + (t - _TAG_START), -1)
    k = 1
    while k < S:
        shifted = jnp.concatenate(
            [jnp.full((Bb, k), -1, jnp.int32), key[:, : S - k]], axis=1
        )
        key = jnp.maximum(key, shifted)
        k *= 2
    nxt = jnp.concatenate([t[:, 1:], jnp.zeros((Bb, 1), jnp.int32)], axis=1)
    qt = jnp.clip(t[:, S - 1 : S] - _TAG_START, 0, _NUM_TAGS - 1)
    task = jnp.clip(t[:, 1:2] - _TASK_START, 0, 3)
    base = (key >= 0) & ((key & 3) == qt)
    v22 = nxt - _VALUE_BASE
    d = nxt + (2 - t * 8)
    m = base & (d >= 0) & (d < 8) & (t >= 3) & (t <= 5)
    pv = jnp.where(m, v22, 24)
    idxs = []
    for flip in (0, 8, 16):
        pf = pv ^ flip
        half = jnp.minimum(pf[:, : S // 2], pf[:, S // 2 :])
        mv = jnp.min(half, axis=1, keepdims=True)
        idxs.append(jnp.where(mv < 8, mv, 0))
    a, b, c = idxs
    affine = (a + 2 * b + 3 * c) & 7
    gate = (a * (b + 1) + c * ((a ^ b) + 1)) & 7
    sel = jnp.where(task == 0, a, jnp.where(task == 1, affine, gate))
    cls = jax.lax.broadcasted_iota(jnp.int32, (Bb, _NUM_CLASSES), 1)
    onehot_logits = jnp.where(cls == sel, _LOGIT_SCALE, 0.0).astype(jnp.float32)
    r = b * 8 + c
    r_iota = jax.lax.broadcasted_iota(jnp.int32, (Bb, 64), 1)
    oh_r = (r_iota == r).astype(jnp.float32)
    p = jnp.dot(oh_r, tbl_ref[...], preferred_element_type=jnp.float32)
    q = jnp.where((r_iota >> 3) == a, p, 0.0)
    q = q[:, :32] + q[:, 32:]
    q = q[:, :16] + q[:, 16:]
    look = q[:, :8] + q[:, 8:]
    out_ref[...] = jnp.where(task == 3, look, onehot_logits)


def _sc_make(Bs, S, row_base):
    info = plsc.get_sparse_core_info()
    NC, NS, L = info.num_cores, info.num_subcores, info.num_lanes
    NW = NC * NS
    rpw = Bs // NW
    G = rpw // L
    mesh = plsc.VectorSubcoreMesh(core_axis_name="c", subcore_axis_name="s")

    @functools.partial(
        pl.kernel,
        mesh=mesh,
        compiler_params=pltpu.CompilerParams(needs_layout_passes=False),
        out_type=jax.ShapeDtypeStruct((Bs, _NUM_CLASSES), jnp.float32),
        scratch_types=[
            pltpu.VMEM((L, S), jnp.int32),
            pltpu.VMEM((4096,), jnp.float32),
            pltpu.VMEM((L, _NUM_CLASSES), jnp.float32),
        ],
    )
    def sc_kernel(tok_hbm, tbl_hbm, out_hbm, buf, tblv, stage):
        wid = lax.axis_index("s") * NC + lax.axis_index("c")
        pltpu.sync_copy(tbl_hbm, tblv)
        row_iota = lax.iota(jnp.int32, L)

        def group_body(gi, _):
            row0 = wid * rpw + gi * L
            pltpu.sync_copy(tok_hbm.at[pl.ds(row_base + row0, L), :], buf)
            col = jnp.zeros((L,), jnp.int32)
            qraw = plsc.load_gather(buf, [row_iota, col + (S - 1)])
            qt = jnp.clip(qraw - _TAG_START, 0, _NUM_TAGS - 1)
            traw = plsc.load_gather(buf, [row_iota, col + 1])
            task = jnp.clip(traw - _TASK_START, 0, 3)
            init = (
                jnp.full((L,), -1, jnp.int32),
                jnp.zeros((L,), jnp.int32),
                jnp.full((L,), 24, jnp.int32),
                jnp.full((L,), 24, jnp.int32),
                jnp.full((L,), 24, jnp.int32),
            )

            def step(s, carry):
                latest, prv, a0, a1, a2 = carry
                g = plsc.load_gather(buf, [row_iota, col + s])
                base = latest == qt
                d = g + (2 - prv * 8)
                m = base & (d >= 0) & (d < 8) & (prv >= 3) & (prv <= 5)
                pv = jnp.where(m, g - _VALUE_BASE, 24)
                a0 = jnp.minimum(a0, pv)
                a1 = jnp.minimum(a1, pv ^ 8)
                a2 = jnp.minimum(a2, pv ^ 16)
                tg = g - _TAG_START
                latest = jnp.where((tg >= 0) & (tg < _NUM_TAGS), tg, latest)
                return (latest, g, a0, a1, a2)

            latest, prv, a0, a1, a2 = lax.fori_loop(0, S, step, init)
            a = jnp.where(a0 < 8, a0, 0)
            b = jnp.where(a1 < 8, a1, 0)
            c = jnp.where(a2 < 8, a2, 0)
            affine = (a + 2 * b + 3 * c) & 7
            gate = (a * (b + 1) + c * ((a ^ b) + 1)) & 7
            sel = jnp.where(task == 0, a, jnp.where(task == 1, affine, gate))
            flat8 = (a * 64 + b * 8 + c) * 8
            for cls in range(_NUM_CLASSES):
                look = plsc.load_gather(tblv, [flat8 + cls])
                oh = jnp.where(sel == cls, _LOGIT_SCALE, 0.0)
                res = jnp.where(task == 3, look, oh)
                plsc.store_scatter(stage, [row_iota, col + cls], res)
            pltpu.sync_copy(stage, out_hbm.at[pl.ds(row0, L), :])
            return 0

        lax.fori_loop(0, G, group_body, 0)

    return sc_kernel


def _tc_call(token_ids, tbl, B_tc, S):
    Bb = min(512, B_tc)
    return pl.pallas_call(
        _body,
        grid=(B_tc // Bb,),
        in_specs=[
            pl.BlockSpec((Bb, S), lambda i: (i, 0)),
            pl.BlockSpec((64, 64), lambda i: (0, 0)),
        ],
        out_specs=pl.BlockSpec((Bb, 8), lambda i: (i, 0)),
        out_shape=jax.ShapeDtypeStruct((B_tc, 8), jnp.float32),
    )(token_ids, tbl)


@jax.jit
def kernel(token_ids, lookup_table):
    B, S = token_ids.shape
    tbl = (
        lookup_table.reshape(8, 64, 8).transpose(1, 0, 2).reshape(64, 64)
    )
    B_sc = 5632 if B >= 16384 else 0
    B_tc = B - B_sc
    out_tc = _tc_call(token_ids, tbl, B_tc, S)
    if B_sc == 0:
        return out_tc
    tblf = lookup_table.reshape(4096)
    out_sc = _sc_make(B_sc, S, B_tc)(token_ids, tblf)
    return jnp.concatenate([out_tc, out_sc], axis=0)


# speedup vs baseline: 1.0708x; 1.0708x over previous
import functools

import jax
import jax.numpy as jnp
from jax import lax
from jax.experimental import pallas as pl
from jax.experimental.pallas import tpu as pltpu
from jax.experimental.pallas import tpu_sc as plsc

_NUM_TAGS = 3
_NUM_CLASSES = 8
_TAG_START = 46
_TASK_START = 49
_VALUE_BASE = 22
_LOGIT_SCALE = 12.0


def _body(tok_ref, tbl_ref, out_ref):
    t = tok_ref[...]
    Bb, S = t.shape
    pos = jax.lax.broadcasted_iota(jnp.int32, (Bb, S), 1)
    is_tag = (t >= _TAG_START) & (t < _TAG_START + _NUM_TAGS)
    key = jnp.where(is_tag, pos * 4 + (t - _TAG_START), -1)
    k = 1
    while k < S:
        shifted = jnp.concatenate(
            [jnp.full((Bb, k), -1, jnp.int32), key[:, : S - k]], axis=1
        )
        key = jnp.maximum(key, shifted)
        k *= 2
    nxt = jnp.concatenate([t[:, 1:], jnp.zeros((Bb, 1), jnp.int32)], axis=1)
    qt = jnp.clip(t[:, S - 1 : S] - _TAG_START, 0, _NUM_TAGS - 1)
    task = jnp.clip(t[:, 1:2] - _TASK_START, 0, 3)
    base = (key >= 0) & ((key & 3) == qt)
    v22 = nxt - _VALUE_BASE
    d = nxt + (2 - t * 8)
    m = base & (d >= 0) & (d < 8) & (t >= 3) & (t <= 5)
    pv = jnp.where(m, v22, 24)
    idxs = []
    for flip in (0, 8, 16):
        pf = pv ^ flip
        half = jnp.minimum(pf[:, : S // 2], pf[:, S // 2 :])
        mv = jnp.min(half, axis=1, keepdims=True)
        idxs.append(jnp.where(mv < 8, mv, 0))
    a, b, c = idxs
    affine = (a + 2 * b + 3 * c) & 7
    gate = (a * (b + 1) + c * ((a ^ b) + 1)) & 7
    sel = jnp.where(task == 0, a, jnp.where(task == 1, affine, gate))
    cls = jax.lax.broadcasted_iota(jnp.int32, (Bb, _NUM_CLASSES), 1)
    onehot_logits = jnp.where(cls == sel, _LOGIT_SCALE, 0.0).astype(jnp.float32)
    flat = a * 64 + b * 8 + c
    f_iota = jax.lax.broadcasted_iota(jnp.int32, (Bb, 512), 1)
    oh = (f_iota == flat).astype(jnp.float32)
    look = jnp.dot(oh, tbl_ref[...], preferred_element_type=jnp.float32)
    out_ref[...] = jnp.where(task == 3, look, onehot_logits)


def _sc_make(Bs, S, row_base):
    info = plsc.get_sparse_core_info()
    NC, NS, L = info.num_cores, info.num_subcores, info.num_lanes
    NW = NC * NS
    rpw = Bs // NW
    G = rpw // L
    mesh = plsc.VectorSubcoreMesh(core_axis_name="c", subcore_axis_name="s")

    @functools.partial(
        pl.kernel,
        mesh=mesh,
        compiler_params=pltpu.CompilerParams(needs_layout_passes=False),
        out_type=jax.ShapeDtypeStruct((Bs, _NUM_CLASSES), jnp.float32),
        scratch_types=[
            pltpu.VMEM((L, S), jnp.int32),
            pltpu.VMEM((4096,), jnp.float32),
            pltpu.VMEM((L, _NUM_CLASSES), jnp.float32),
        ],
    )
    def sc_kernel(tok_hbm, tbl_hbm, out_hbm, buf, tblv, stage):
        wid = lax.axis_index("s") * NC + lax.axis_index("c")
        pltpu.sync_copy(tbl_hbm, tblv)
        row_iota = lax.iota(jnp.int32, L)

        def group_body(gi, _):
            row0 = wid * rpw + gi * L
            pltpu.sync_copy(tok_hbm.at[pl.ds(row_base + row0, L), :], buf)
            col = jnp.zeros((L,), jnp.int32)
            qraw = plsc.load_gather(buf, [row_iota, col + (S - 1)])
            qt = jnp.clip(qraw - _TAG_START, 0, _NUM_TAGS - 1)
            traw = plsc.load_gather(buf, [row_iota, col + 1])
            task = jnp.clip(traw - _TASK_START, 0, 3)
            init = (
                jnp.full((L,), -1, jnp.int32),
                jnp.zeros((L,), jnp.int32),
                jnp.full((L,), 24, jnp.int32),
                jnp.full((L,), 24, jnp.int32),
                jnp.full((L,), 24, jnp.int32),
            )

            def step(s, carry):
                latest, prv, a0, a1, a2 = carry
                g = plsc.load_gather(buf, [row_iota, col + s])
                base = latest == qt
                d = g + (2 - prv * 8)
                m = base & (d >= 0) & (d < 8) & (prv >= 3) & (prv <= 5)
                pv = jnp.where(m, g - _VALUE_BASE, 24)
                a0 = jnp.minimum(a0, pv)
                a1 = jnp.minimum(a1, pv ^ 8)
                a2 = jnp.minimum(a2, pv ^ 16)
                tg = g - _TAG_START
                latest = jnp.where((tg >= 0) & (tg < _NUM_TAGS), tg, latest)
                return (latest, g, a0, a1, a2)

            latest, prv, a0, a1, a2 = lax.fori_loop(0, S, step, init)
            a = jnp.where(a0 < 8, a0, 0)
            b = jnp.where(a1 < 8, a1, 0)
            c = jnp.where(a2 < 8, a2, 0)
            affine = (a + 2 * b + 3 * c) & 7
            gate = (a * (b + 1) + c * ((a ^ b) + 1)) & 7
            sel = jnp.where(task == 0, a, jnp.where(task == 1, affine, gate))
            flat8 = (a * 64 + b * 8 + c) * 8
            for cls in range(_NUM_CLASSES):
                look = plsc.load_gather(tblv, [flat8 + cls])
                oh = jnp.where(sel == cls, _LOGIT_SCALE, 0.0)
                res = jnp.where(task == 3, look, oh)
                plsc.store_scatter(stage, [row_iota, col + cls], res)
            pltpu.sync_copy(stage, out_hbm.at[pl.ds(row0, L), :])
            return 0

        lax.fori_loop(0, G, group_body, 0)

    return sc_kernel


def _tc_call(token_ids, tbl, B_tc, S):
    Bb = min(512, B_tc)
    return pl.pallas_call(
        _body,
        grid=(B_tc // Bb,),
        in_specs=[
            pl.BlockSpec((Bb, S), lambda i: (i, 0)),
            pl.BlockSpec((512, 8), lambda i: (0, 0)),
        ],
        out_specs=pl.BlockSpec((Bb, 8), lambda i: (i, 0)),
        out_shape=jax.ShapeDtypeStruct((B_tc, 8), jnp.float32),
    )(token_ids, tbl)


@jax.jit
def kernel(token_ids, lookup_table):
    B, S = token_ids.shape
    tbl = lookup_table.reshape(512, 8)
    B_sc = 5632 if B >= 16384 else 0
    B_tc = B - B_sc
    out_tc = _tc_call(token_ids, tbl, B_tc, S)
    if B_sc == 0:
        return out_tc
    tblf = lookup_table.reshape(4096)
    out_sc = _sc_make(B_sc, S, B_tc)(token_ids, tblf)
    return jnp.concatenate([out_tc, out_sc], axis=0)
